# Initial kernel scaffold; baseline (speedup 1.0000x reference)
import jax
import jax.numpy as jnp
from jax.experimental import pallas as pl


def kernel(row_embeddings, column_embeddings):
    raise NotImplementedError("write your pallas kernel here")


# speedup vs baseline: 1.0374x; 1.0374x over previous
import jax
import jax.numpy as jnp
from jax.experimental import pallas as pl

BATCH = 64
HW = 32
D = 256

BPB = 4


def _body(row_ref, col_ref, out_ref):
    col = col_ref[...]
    row = row_ref[...]
    x = jax.lax.broadcast_in_dim(col, (BPB, HW, HW, D), (2, 3))
    y = jax.lax.broadcast_in_dim(row, (BPB, HW, HW, D), (1, 3))
    out_ref[:, :, :, 0:D] = x
    out_ref[:, :, :, D : 2 * D] = y


def kernel(row_embeddings, column_embeddings):
    row = row_embeddings[:HW]
    col = column_embeddings[:HW]

    out4 = pl.pallas_call(
        _body,
        grid=(BATCH // BPB,),
        in_specs=[
            pl.BlockSpec((HW, D), lambda b: (0, 0)),
            pl.BlockSpec((HW, D), lambda b: (0, 0)),
        ],
        out_specs=pl.BlockSpec((BPB, HW, HW, 2 * D), lambda b: (b, 0, 0, 0)),
        out_shape=jax.ShapeDtypeStruct((BATCH, HW, HW, 2 * D), jnp.float32),
    )(row, col)
    return out4.reshape(BATCH, HW * HW, 2 * D)
